# baseline (speedup 1.0000x reference)
import jax
import jax.numpy as jnp
from jax.experimental import pallas as pl
from jax.experimental.pallas import tpu as pltpu

_NUM_EXPERTS = 16
_CAPACITY = 512.0
_SUB = 256


def _router_body(x_ref, w_ref, ei_ref, pm_ref, lg_ref, carry_ref, tri_ref):
    g = pl.program_id(0)
    b = pl.program_id(1)

    @pl.when((g == 0) & (b == 0))
    def _():
        r = jax.lax.broadcasted_iota(jnp.int32, (_SUB, _SUB), 0)
        c = jax.lax.broadcasted_iota(jnp.int32, (_SUB, _SUB), 1)
        tri_ref[...] = (r >= c).astype(jnp.bfloat16)

    @pl.when(b == 0)
    def _():
        carry_ref[...] = jnp.zeros_like(carry_ref)

    x = x_ref[0]
    w = w_ref[...]
    logits = jnp.dot(x, w, preferred_element_type=jnp.float32)
    lg_ref[0] = logits

    m = jnp.max(logits, axis=-1, keepdims=True)
    s = jnp.sum(jnp.exp(logits - m), axis=-1, keepdims=True)
    pm_ref[0] = 1.0 / s

    bt = logits.shape[0]
    E = logits.shape[1]
    eq = logits == m
    er = jax.lax.broadcasted_iota(jnp.int32, (E, E), 0)
    ec = jax.lax.broadcasted_iota(jnp.int32, (E, E), 1)
    lmat = (er < ec).astype(jnp.bfloat16)
    before = jnp.dot(eq.astype(jnp.bfloat16), lmat,
                     preferred_element_type=jnp.float32)
    sel = eq & (before == 0.0)
    oh = sel.astype(jnp.bfloat16)

    tri = tri_ref[...]
    for si in range(bt // _SUB):
        lo = si * _SUB
        csum = jnp.dot(tri, oh[lo:lo + _SUB, :],
                       preferred_element_type=jnp.float32)
        prio = csum + carry_ref[...]
        carry_ref[...] = carry_ref[...] + csum[_SUB - 1:_SUB, :]
        keep = prio <= _CAPACITY
        ei_ref[0, lo:lo + _SUB, :] = (
            sel[lo:lo + _SUB, :] & keep).astype(jnp.int32)


def kernel(hidden_states, W):
    G, T, H = hidden_states.shape
    E = W.shape[1]
    BT = 1024
    nb = T // BT

    grid = (G, nb)
    out_shapes = (
        jax.ShapeDtypeStruct((G, T, E), jnp.int32),
        jax.ShapeDtypeStruct((G, T, 1), jnp.float32),
        jax.ShapeDtypeStruct((G, T, E), jnp.float32),
    )
    out_specs = (
        pl.BlockSpec((1, BT, E), lambda g, b: (g, b, 0)),
        pl.BlockSpec((1, BT, 1), lambda g, b: (g, b, 0)),
        pl.BlockSpec((1, BT, E), lambda g, b: (g, b, 0)),
    )
    in_specs = (
        pl.BlockSpec((1, BT, H), lambda g, b: (g, b, 0)),
        pl.BlockSpec((H, E), lambda g, b: (0, 0)),
    )

    return pl.pallas_call(
        _router_body,
        grid=grid,
        in_specs=in_specs,
        out_specs=out_specs,
        out_shape=out_shapes,
        scratch_shapes=[
            pltpu.VMEM((1, E), jnp.float32),
            pltpu.VMEM((_SUB, _SUB), jnp.bfloat16),
        ],
        compiler_params=pltpu.CompilerParams(
            dimension_semantics=("arbitrary", "arbitrary"),
        ),
    )(hidden_states, W)
